# baseline (speedup 1.0000x reference)
import jax
import jax.numpy as jnp
from jax.experimental import pallas as pl
from jax.experimental.pallas import tpu as pltpu
from jax.experimental.pallas import tpu_sc as plsc

NUM_EXPERTS = 8
TOP_K = 2
TILE = 4096
LANES = 16
BLK = 128


def _probs_kernel(x_ref, w_ref, p_ref):
    x = x_ref[...]
    w = w_ref[...]
    logits = jax.lax.dot_general(
        w, x, (((1,), (1,)), ((), ())),
        preferred_element_type=jnp.float32)
    m = jnp.max(logits, axis=0, keepdims=True)
    e = jnp.exp(logits - m)
    p_ref[...] = e / jnp.sum(e, axis=0, keepdims=True)


def _tc_probs(hidden_states, W):
    n, d = hidden_states.shape
    return pl.pallas_call(
        _probs_kernel,
        grid=(n // TILE,),
        in_specs=[
            pl.BlockSpec((TILE, d), lambda i: (i, 0)),
            pl.BlockSpec((NUM_EXPERTS, d), lambda i: (0, 0)),
        ],
        out_specs=pl.BlockSpec((NUM_EXPERTS, TILE), lambda i: (0, i)),
        out_shape=jax.ShapeDtypeStruct((NUM_EXPERTS, n), jnp.float32),
        compiler_params=pltpu.CompilerParams(
            dimension_semantics=("parallel",),
        ),
    )(hidden_states, W)


def _sc_top2(probs_t):
    n = probs_t.shape[1]
    vector_mesh = plsc.VectorSubcoreMesh(
        core_axis_name="core", subcore_axis_name="subcore")

    @pl.kernel(
        out_type=[
            jax.ShapeDtypeStruct((TOP_K, n), jnp.float32),
            jax.ShapeDtypeStruct((TOP_K, n), jnp.int32),
        ],
        mesh=vector_mesh,
        scratch_types=[],
    )
    def sc_kernel(x_hbm_ref, p_hbm_ref, i_hbm_ref):
        def body(in_vmem, p_vmem, i_vmem):
            @pl.loop(0, BLK, step=LANES)
            def _(c1):
                slc = pl.ds(c1, LANES)
                rows = [in_vmem.at[e, slc][...] for e in range(NUM_EXPERTS)]
                v1 = rows[0]
                i1 = jnp.zeros((LANES,), jnp.int32)
                for e in range(1, NUM_EXPERTS):
                    m = rows[e] > v1
                    i1 = jnp.where(m, e, i1)
                    v1 = jnp.where(m, rows[e], v1)
                v2 = jnp.full((LANES,), -1.0, jnp.float32)
                i2 = jnp.zeros((LANES,), jnp.int32)
                for e in range(NUM_EXPERTS):
                    m = (rows[e] > v2) & (i1 != e)
                    i2 = jnp.where(m, e, i2)
                    v2 = jnp.where(m, rows[e], v2)
                p_vmem.at[0, slc][...] = v1
                p_vmem.at[1, slc][...] = v2
                i_vmem.at[0, slc][...] = i1
                i_vmem.at[1, slc][...] = i2

        pltpu.emit_pipeline(
            body,
            grid=(n // BLK,),
            in_specs=[pl.BlockSpec((NUM_EXPERTS, BLK), lambda i: (0, i))],
            out_specs=[
                pl.BlockSpec((TOP_K, BLK), lambda i: (0, i)),
                pl.BlockSpec((TOP_K, BLK), lambda i: (0, i)),
            ],
            core_axis_name=("core", "subcore"),
            dimension_semantics=(pltpu.PARALLEL,),
        )(x_hbm_ref, p_hbm_ref, i_hbm_ref)

    return sc_kernel(probs_t)


def kernel(hidden_states, W):
    probs_t = _tc_probs(hidden_states, W)
    p_t, i_t = _sc_top2(probs_t)
    return (p_t.T, i_t.T)


# speedup vs baseline: 1.6163x; 1.6163x over previous
import jax
import jax.numpy as jnp
from jax.experimental import pallas as pl
from jax.experimental.pallas import tpu as pltpu

NUM_EXPERTS = 8
TOP_K = 2
TILE = 4096


def _router_kernel(x_ref, w_ref, p_ref, i_ref):
    x = x_ref[...]
    w = w_ref[...]
    logits = jax.lax.dot_general(
        w, x, (((1,), (1,)), ((), ())),
        preferred_element_type=jnp.float32)

    m = jnp.max(logits, axis=0, keepdims=True)
    e = jnp.exp(logits - m)
    probs = e / jnp.sum(e, axis=0, keepdims=True)

    i1 = jnp.argmax(probs, axis=0)
    p1 = jnp.max(probs, axis=0)
    row = jax.lax.broadcasted_iota(jnp.int32, probs.shape, 0)
    masked = jnp.where(row == i1[None, :], -1.0, probs)
    i2 = jnp.argmax(masked, axis=0)
    p2 = jnp.max(masked, axis=0)

    p_ref[...] = jnp.concatenate([p1[None, :], p2[None, :]], axis=0)
    i_ref[...] = jnp.concatenate([i1[None, :], i2[None, :]], axis=0).astype(jnp.int32)


def kernel(hidden_states, W):
    n, d = hidden_states.shape
    probs_t, idx_t = pl.pallas_call(
        _router_kernel,
        grid=(n // TILE,),
        in_specs=[
            pl.BlockSpec((TILE, d), lambda i: (i, 0)),
            pl.BlockSpec((NUM_EXPERTS, d), lambda i: (0, 0)),
        ],
        out_specs=[
            pl.BlockSpec((TOP_K, TILE), lambda i: (0, i)),
            pl.BlockSpec((TOP_K, TILE), lambda i: (0, i)),
        ],
        out_shape=[
            jax.ShapeDtypeStruct((TOP_K, n), jnp.float32),
            jax.ShapeDtypeStruct((TOP_K, n), jnp.int32),
        ],
        compiler_params=pltpu.CompilerParams(
            dimension_semantics=("parallel",),
        ),
    )(hidden_states, W)
    return (probs_t.T, idx_t.T)
